# baseline (speedup 1.0000x reference)
import functools

import jax
import jax.numpy as jnp
from jax.experimental import pallas as pl
from jax.experimental.pallas import tpu as pltpu


def _fused_body(xl_ref, xh_ref, fl_ref, fh_ref, w_ref, b_ref,
                y_ref, stats_ref, sum_acc, sq_acc):
    xl = xl_ref[0]
    xh = xh_ref[0]
    nh = xh.shape[0]

    lt = xl.shape[0]
    ql2 = jnp.sum(xl * xl, axis=1)[None, :]
    h2 = jnp.sum(xh * xh, axis=1, keepdims=True)
    cross = jax.lax.dot_general(xh, xl, (((1,), (1,)), ((), ())),
                                preferred_element_type=jnp.float32)
    d2 = h2 + ql2 - 2.0 * cross

    big = jnp.float32(3.0e38)
    m0 = jnp.full((8, lt), big, jnp.float32)
    m1 = m0
    m2 = m0
    for s in range(nh // 8):
        v = d2[s * 8:(s + 1) * 8, :]
        h0 = jnp.maximum(m0, v)
        m0 = jnp.minimum(m0, v)
        h1 = jnp.maximum(m1, h0)
        m1 = jnp.minimum(m1, h0)
        m2 = jnp.minimum(m2, h1)

    cand = jnp.concatenate([m0, m1, m2], axis=0)
    ci = jax.lax.broadcasted_iota(jnp.int32, cand.shape, 0)
    for _ in range(2):
        mk = jnp.min(cand, axis=0, keepdims=True)
        ik = jnp.min(jnp.where(cand == mk, ci, 24), axis=0, keepdims=True)
        cand = jnp.where(ci == ik, big, cand)
    thresh = jnp.min(cand, axis=0, keepdims=True)

    su = jnp.where(d2 <= thresh,
                   jax.lax.rsqrt(jnp.maximum(d2, 1e-12)),
                   0.0)
    wsum = jnp.sum(su, axis=0, keepdims=True)

    fh = fh_ref[0]
    fi_un = jax.lax.dot_general(fh, su.astype(jnp.bfloat16),
                                (((1,), (0,)), ((), ())),
                                preferred_element_type=jnp.float32)
    fi = fi_un * (1.0 / wsum)
    fcat = jnp.concatenate([fi.astype(jnp.bfloat16), fl_ref[0]], axis=0)
    y = jax.lax.dot_general(w_ref[...], fcat, (((1,), (0,)), ((), ())),
                            preferred_element_type=jnp.float32) + b_ref[...]
    y_ref[0] = y

    ys = jnp.sum(y, axis=1, keepdims=True)
    ysq = jnp.sum(y * y, axis=1, keepdims=True)

    step = pl.program_id(0) * pl.num_programs(1) + pl.program_id(1)

    @pl.when(step == 0)
    def _():
        sum_acc[...] = ys
        sq_acc[...] = ysq

    @pl.when(step > 0)
    def _():
        sum_acc[...] += ys
        sq_acc[...] += ysq

    @pl.when(step == pl.num_programs(0) * pl.num_programs(1) - 1)
    def _():
        stats_ref[...] = jnp.concatenate([sum_acc[...], sq_acc[...]], axis=1)


def _norm_body(y_ref, stats_ref, gamma_ref, beta_ref, o_ref, *, inv_n):
    mean = stats_ref[:, 0:1] * inv_n
    msq = stats_ref[:, 1:2] * inv_n
    var = msq - mean * mean
    rstd = jax.lax.rsqrt(var + 1e-5)
    scale = gamma_ref[...] * rstd
    shift = beta_ref[...] - mean * scale
    o_ref[0] = jnp.maximum(y_ref[0] * scale + shift, 0.0)


def kernel(xyz_low, xyz_high, feat_low, feat_high, W, b, gamma, beta):
    B, Nl, _ = xyz_low.shape
    Nh = xyz_high.shape[1]
    Cl = feat_low.shape[1]
    Ch = feat_high.shape[1]
    out_ch = W.shape[0]
    Lt = 4096
    grid = (B, Nl // Lt)

    y, stats = pl.pallas_call(
        _fused_body,
        grid=grid,
        in_specs=[
            pl.BlockSpec((1, Lt, 3), lambda bi, li: (bi, li, 0)),
            pl.BlockSpec((1, Nh, 3), lambda bi, li: (bi, 0, 0)),
            pl.BlockSpec((1, Cl, Lt), lambda bi, li: (bi, 0, li)),
            pl.BlockSpec((1, Ch, Nh), lambda bi, li: (bi, 0, 0)),
            pl.BlockSpec((out_ch, Cl + Ch), lambda bi, li: (0, 0)),
            pl.BlockSpec((out_ch, 1), lambda bi, li: (0, 0)),
        ],
        out_specs=[
            pl.BlockSpec((1, out_ch, Lt), lambda bi, li: (bi, 0, li)),
            pl.BlockSpec((out_ch, 2), lambda bi, li: (0, 0)),
        ],
        out_shape=[
            jax.ShapeDtypeStruct((B, out_ch, Nl), jnp.float32),
            jax.ShapeDtypeStruct((out_ch, 2), jnp.float32),
        ],
        scratch_shapes=[
            pltpu.VMEM((out_ch, 1), jnp.float32),
            pltpu.VMEM((out_ch, 1), jnp.float32),
        ],
    )(xyz_low, xyz_high, feat_low.astype(jnp.bfloat16),
      feat_high.astype(jnp.bfloat16), W.astype(jnp.bfloat16),
      b.reshape(out_ch, 1))

    Ln = 1024
    out = pl.pallas_call(
        functools.partial(_norm_body, inv_n=1.0 / (B * Nl)),
        grid=(B, Nl // Ln),
        in_specs=[
            pl.BlockSpec((1, out_ch, Ln), lambda bi, li: (bi, 0, li)),
            pl.BlockSpec((out_ch, 2), lambda bi, li: (0, 0)),
            pl.BlockSpec((out_ch, 1), lambda bi, li: (0, 0)),
            pl.BlockSpec((out_ch, 1), lambda bi, li: (0, 0)),
        ],
        out_specs=pl.BlockSpec((1, out_ch, Ln), lambda bi, li: (bi, 0, li)),
        out_shape=jax.ShapeDtypeStruct((B, out_ch, Nl), jnp.float32),
    )(y, stats, gamma.reshape(out_ch, 1), beta.reshape(out_ch, 1))
    return out


# speedup vs baseline: 1.1082x; 1.1082x over previous
import functools

import jax
import jax.numpy as jnp
from jax.experimental import pallas as pl
from jax.experimental.pallas import tpu as pltpu


def _fused_body(xl_ref, xh_ref, fl_ref, fh_ref, w_ref, b_ref,
                y_ref, stats_ref, sum_acc, sq_acc):
    xl = xl_ref[0]
    xh = xh_ref[0]
    nh = xh.shape[0]

    lt = xl.shape[0]
    ql2 = jnp.sum(xl * xl, axis=1)[None, :]
    h2 = jnp.sum(xh * xh, axis=1, keepdims=True)
    cross = jax.lax.dot_general(xh, xl, (((1,), (1,)), ((), ())),
                                preferred_element_type=jnp.float32)
    d2 = h2 + ql2 - 2.0 * cross

    big = jnp.float32(3.0e38)
    m0 = jnp.full((8, lt), big, jnp.float32)
    m1 = m0
    m2 = m0
    for s in range(nh // 8):
        v = d2[s * 8:(s + 1) * 8, :]
        h0 = jnp.maximum(m0, v)
        m0 = jnp.minimum(m0, v)
        h1 = jnp.maximum(m1, h0)
        m1 = jnp.minimum(m1, h0)
        m2 = jnp.minimum(m2, h1)

    cand = jnp.concatenate([m0, m1, m2], axis=0)
    ci = jax.lax.broadcasted_iota(jnp.int32, cand.shape, 0)
    for _ in range(2):
        mk = jnp.min(cand, axis=0, keepdims=True)
        ik = jnp.min(jnp.where(cand == mk, ci, 24), axis=0, keepdims=True)
        cand = jnp.where(ci == ik, big, cand)
    thresh = jnp.min(cand, axis=0, keepdims=True)

    su = jnp.where(d2 <= thresh,
                   jax.lax.rsqrt(jnp.maximum(d2, 1e-12)),
                   0.0)
    wsum = jnp.sum(su, axis=0, keepdims=True)

    fh = fh_ref[0].astype(jnp.bfloat16)
    fi_un = jax.lax.dot_general(fh, su.astype(jnp.bfloat16),
                                (((1,), (0,)), ((), ())),
                                preferred_element_type=jnp.float32)
    fi = fi_un * (1.0 / wsum)
    fcat = jnp.concatenate([fi.astype(jnp.bfloat16),
                            fl_ref[0].astype(jnp.bfloat16)], axis=0)
    y = jax.lax.dot_general(w_ref[...].astype(jnp.bfloat16), fcat,
                            (((1,), (0,)), ((), ())),
                            preferred_element_type=jnp.float32) + b_ref[...]
    y_ref[0] = y.astype(jnp.bfloat16)

    ys = jnp.sum(y, axis=1, keepdims=True)
    ysq = jnp.sum(y * y, axis=1, keepdims=True)

    step = pl.program_id(0) * pl.num_programs(1) + pl.program_id(1)

    @pl.when(step == 0)
    def _():
        sum_acc[...] = ys
        sq_acc[...] = ysq

    @pl.when(step > 0)
    def _():
        sum_acc[...] += ys
        sq_acc[...] += ysq

    @pl.when(step == pl.num_programs(0) * pl.num_programs(1) - 1)
    def _():
        stats_ref[...] = jnp.concatenate([sum_acc[...], sq_acc[...]], axis=1)


def _norm_body(y_ref, stats_ref, gamma_ref, beta_ref, o_ref, *, inv_n):
    mean = stats_ref[:, 0:1] * inv_n
    msq = stats_ref[:, 1:2] * inv_n
    var = msq - mean * mean
    rstd = jax.lax.rsqrt(var + 1e-5)
    scale = gamma_ref[...] * rstd
    shift = beta_ref[...] - mean * scale
    o_ref[0] = jnp.maximum(y_ref[0].astype(jnp.float32) * scale + shift, 0.0)


def kernel(xyz_low, xyz_high, feat_low, feat_high, W, b, gamma, beta):
    B, Nl, _ = xyz_low.shape
    Nh = xyz_high.shape[1]
    Cl = feat_low.shape[1]
    Ch = feat_high.shape[1]
    out_ch = W.shape[0]
    Lt = 2048
    grid = (B, Nl // Lt)

    y, stats = pl.pallas_call(
        _fused_body,
        grid=grid,
        in_specs=[
            pl.BlockSpec((1, Lt, 3), lambda bi, li: (bi, li, 0)),
            pl.BlockSpec((1, Nh, 3), lambda bi, li: (bi, 0, 0)),
            pl.BlockSpec((1, Cl, Lt), lambda bi, li: (bi, 0, li)),
            pl.BlockSpec((1, Ch, Nh), lambda bi, li: (bi, 0, 0)),
            pl.BlockSpec((out_ch, Cl + Ch), lambda bi, li: (0, 0)),
            pl.BlockSpec((out_ch, 1), lambda bi, li: (0, 0)),
        ],
        out_specs=[
            pl.BlockSpec((1, out_ch, Lt), lambda bi, li: (bi, 0, li)),
            pl.BlockSpec((out_ch, 2), lambda bi, li: (0, 0)),
        ],
        out_shape=[
            jax.ShapeDtypeStruct((B, out_ch, Nl), jnp.bfloat16),
            jax.ShapeDtypeStruct((out_ch, 2), jnp.float32),
        ],
        scratch_shapes=[
            pltpu.VMEM((out_ch, 1), jnp.float32),
            pltpu.VMEM((out_ch, 1), jnp.float32),
        ],
    )(xyz_low, xyz_high, feat_low, feat_high, W, b.reshape(out_ch, 1))

    Ln = 1024
    out = pl.pallas_call(
        functools.partial(_norm_body, inv_n=1.0 / (B * Nl)),
        grid=(B, Nl // Ln),
        in_specs=[
            pl.BlockSpec((1, out_ch, Ln), lambda bi, li: (bi, 0, li)),
            pl.BlockSpec((out_ch, 2), lambda bi, li: (0, 0)),
            pl.BlockSpec((out_ch, 1), lambda bi, li: (0, 0)),
            pl.BlockSpec((out_ch, 1), lambda bi, li: (0, 0)),
        ],
        out_specs=pl.BlockSpec((1, out_ch, Ln), lambda bi, li: (bi, 0, li)),
        out_shape=jax.ShapeDtypeStruct((B, out_ch, Nl), jnp.float32),
    )(y, stats, gamma.reshape(out_ch, 1), beta.reshape(out_ch, 1))
    return out
